# baseline (speedup 1.0000x reference)
import functools

import jax
import jax.numpy as jnp
from jax.experimental import pallas as pl

HIDDEN = 768
INTER = 2048
NUM_EXPERTS = 64

ROUTER_BLOCK = 2048
FFN_IC = 512


def _router_body(x_ref, gw_ref, logits_ref, m_ref, s_ref, aux_ref):
    i = pl.program_id(0)

    @pl.when(i == 0)
    def _init():
        m_ref[...] = jnp.full_like(m_ref, -1e30)
        s_ref[...] = jnp.zeros_like(s_ref)
        aux_ref[...] = jnp.zeros_like(aux_ref)

    xb = x_ref[...]
    gw = gw_ref[...]
    lg = jax.lax.dot_general(xb, gw, (((1,), (1,)), ((), ())),
                             preferred_element_type=jnp.float32)
    logits_ref[...] = lg

    blkmax = jnp.max(lg, axis=0, keepdims=True)
    m_old = m_ref[...]
    m_new = jnp.maximum(m_old, blkmax)
    s_ref[...] = (s_ref[...] * jnp.exp(m_old - m_new)
                  + jnp.sum(jnp.exp(lg - m_new), axis=0, keepdims=True))
    m_ref[...] = m_new

    tmax = jnp.max(lg, axis=1, keepdims=True)
    lse = jnp.log(jnp.sum(jnp.exp(lg - tmax), axis=1, keepdims=True)) + tmax
    aux_ref[...] += jnp.sum(lse * lse).reshape(1, 1)


def _router(x, gate_w):
    n, h = x.shape
    e = gate_w.shape[0]
    nblocks = n // ROUTER_BLOCK
    return pl.pallas_call(
        _router_body,
        grid=(nblocks,),
        in_specs=[
            pl.BlockSpec((ROUTER_BLOCK, h), lambda i: (i, 0)),
            pl.BlockSpec((e, h), lambda i: (0, 0)),
        ],
        out_specs=[
            pl.BlockSpec((ROUTER_BLOCK, e), lambda i: (i, 0)),
            pl.BlockSpec((1, e), lambda i: (0, 0)),
            pl.BlockSpec((1, e), lambda i: (0, 0)),
            pl.BlockSpec((1, 1), lambda i: (0, 0)),
        ],
        out_shape=[
            jax.ShapeDtypeStruct((n, e), jnp.float32),
            jax.ShapeDtypeStruct((1, e), jnp.float32),
            jax.ShapeDtypeStruct((1, e), jnp.float32),
            jax.ShapeDtypeStruct((1, 1), jnp.float32),
        ],
    )(x, gate_w)


def _ffn_body(nk, xe_ref, wg_ref, wu_ref, wd_ref, p_ref, out_ref):
    k = pl.program_id(1)
    xe = xe_ref[0].astype(jnp.bfloat16)
    wg = wg_ref[0].astype(jnp.bfloat16)
    wu = wu_ref[0].astype(jnp.bfloat16)
    g = jax.lax.dot_general(xe, wg, (((1,), (1,)), ((), ())),
                            preferred_element_type=jnp.float32)
    u = jax.lax.dot_general(xe, wu, (((1,), (1,)), ((), ())),
                            preferred_element_type=jnp.float32)
    hact = ((g * jax.nn.sigmoid(g)) * u).astype(jnp.bfloat16)
    wd = wd_ref[0].astype(jnp.bfloat16)
    o = jax.lax.dot_general(hact, wd, (((1,), (1,)), ((), ())),
                            preferred_element_type=jnp.float32)

    @pl.when(k == 0)
    def _():
        out_ref[0] = o

    @pl.when(k > 0)
    def _():
        out_ref[0] += o

    @pl.when(k == nk - 1)
    def _():
        out_ref[0] *= p_ref[0]


def _ffn(expert_in, gate_proj_w, up_proj_w, down_proj_w, top_probs):
    e, cap, h = expert_in.shape
    inter = gate_proj_w.shape[1]
    nk = inter // FFN_IC
    return pl.pallas_call(
        functools.partial(_ffn_body, nk),
        grid=(e, nk),
        in_specs=[
            pl.BlockSpec((1, cap, h), lambda ei, k: (ei, 0, 0)),
            pl.BlockSpec((1, FFN_IC, h), lambda ei, k: (ei, k, 0)),
            pl.BlockSpec((1, FFN_IC, h), lambda ei, k: (ei, k, 0)),
            pl.BlockSpec((1, h, FFN_IC), lambda ei, k: (ei, 0, k)),
            pl.BlockSpec((1, cap, 1), lambda ei, k: (ei, 0, 0)),
        ],
        out_specs=pl.BlockSpec((1, cap, h), lambda ei, k: (ei, 0, 0)),
        out_shape=jax.ShapeDtypeStruct((e, cap, h), jnp.float32),
    )(expert_in, gate_proj_w, up_proj_w, down_proj_w, top_probs[..., None])


def kernel(hidden_states, gate_w, gate_proj_w, up_proj_w, down_proj_w):
    b, seq, h = hidden_states.shape
    x = hidden_states.reshape(-1, h)
    n = x.shape[0]
    e = gate_w.shape[0]
    cap = max(n // e, 1)
    cap = min(cap, n)

    logits, m, s, aux_sum = _router(x, gate_w)

    top_logits, top_idx = jax.lax.top_k(logits.T, cap)
    top_probs = jnp.exp(top_logits - m.reshape(e, 1)) / s.reshape(e, 1)

    expert_in = x[top_idx]
    weighted = _ffn(expert_in, gate_proj_w, up_proj_w, down_proj_w, top_probs)

    flat_idx = top_idx.reshape(-1)
    final = jnp.zeros_like(x).at[flat_idx].add(weighted.reshape(-1, h))
    token_counts = jnp.zeros((n,), x.dtype).at[flat_idx].add(top_probs.reshape(-1))
    token_counts = jnp.clip(token_counts, 1e-9, None)
    final = (final / token_counts[:, None]).reshape(b, seq, h)

    aux_loss = (aux_sum.reshape(()) / n) * 0.001
    return final, aux_loss


# speedup vs baseline: 1.6813x; 1.6813x over previous
import functools

import jax
import jax.numpy as jnp
from jax import lax
from jax.experimental import pallas as pl
from jax.experimental.pallas import tpu as pltpu
from jax.experimental.pallas import tpu_sc as plsc

HIDDEN = 768
INTER = 2048
NUM_EXPERTS = 64
N_TOKENS = 32768
CAP = 512

ROUTER_BLOCK = 2048
FFN_IC = 512

SC_CORES = 2
SC_SUBCORES = 16
SC_WORKERS = SC_CORES * SC_SUBCORES
EXP_PER_WORKER = NUM_EXPERTS // SC_WORKERS
LOGIT_CHUNK = 8192
GATHER_CHUNK = 64


def _router_body(x_ref, gw_ref, logits_ref, m_ref, s_ref, aux_ref):
    i = pl.program_id(0)

    @pl.when(i == 0)
    def _init():
        m_ref[...] = jnp.full_like(m_ref, -1e30)
        s_ref[...] = jnp.zeros_like(s_ref)
        aux_ref[...] = jnp.zeros_like(aux_ref)

    xb = x_ref[...]
    gw = gw_ref[...]
    lg = jax.lax.dot_general(gw, xb, (((1,), (1,)), ((), ())),
                             preferred_element_type=jnp.float32)
    logits_ref[...] = lg

    blkmax = jnp.max(lg, axis=1, keepdims=True)
    m_old = m_ref[...]
    m_new = jnp.maximum(m_old, blkmax)
    s_ref[...] = (s_ref[...] * jnp.exp(m_old - m_new)
                  + jnp.sum(jnp.exp(lg - m_new), axis=1, keepdims=True))
    m_ref[...] = m_new

    tmax = jnp.max(lg, axis=0, keepdims=True)
    lse = jnp.log(jnp.sum(jnp.exp(lg - tmax), axis=0, keepdims=True)) + tmax
    aux_ref[...] += jnp.sum(lse * lse).reshape(1, 1)


def _router(x, gate_w):
    n, h = x.shape
    e = gate_w.shape[0]
    nblocks = n // ROUTER_BLOCK
    return pl.pallas_call(
        _router_body,
        grid=(nblocks,),
        in_specs=[
            pl.BlockSpec((ROUTER_BLOCK, h), lambda i: (i, 0)),
            pl.BlockSpec((e, h), lambda i: (0, 0)),
        ],
        out_specs=[
            pl.BlockSpec((e, ROUTER_BLOCK), lambda i: (0, i)),
            pl.BlockSpec((e, 1), lambda i: (0, 0)),
            pl.BlockSpec((e, 1), lambda i: (0, 0)),
            pl.BlockSpec((1, 1), lambda i: (0, 0)),
        ],
        out_shape=[
            jax.ShapeDtypeStruct((e, n), jnp.float32),
            jax.ShapeDtypeStruct((e, 1), jnp.float32),
            jax.ShapeDtypeStruct((e, 1), jnp.float32),
            jax.ShapeDtypeStruct((1, 1), jnp.float32),
        ],
    )(x, gate_w)


def _bisect_body(cap, lgt_ref, m_ref, s_ref, params_ref):
    lg = lgt_ref[...]
    bits = jax.lax.bitcast_convert_type(lg, jnp.int32)
    keys = jnp.where(bits >= 0, bits, bits ^ jnp.int32(0x7FFFFFFF))

    e = lg.shape[0]
    imin = jnp.iinfo(jnp.int32).min
    imax = jnp.iinfo(jnp.int32).max
    lo0 = jnp.full((e, 1), imin, jnp.int32)
    hi0 = jnp.full((e, 1), imax, jnp.int32)

    def body(_, c):
        lo, hi = c
        fl = (lo & hi) + ((lo ^ hi) >> 1)
        mid = fl + ((lo ^ hi) & 1)
        cnt = jnp.sum((keys >= mid).astype(jnp.int32), axis=1, keepdims=True)
        pred = cnt >= cap
        return jnp.where(pred, mid, lo), jnp.where(pred, hi, mid - 1)

    lo, _ = jax.lax.fori_loop(0, 32, body, (lo0, hi0))
    tbits = jnp.where(lo >= 0, lo, lo ^ jnp.int32(0x7FFFFFFF))
    thresh = jax.lax.bitcast_convert_type(tbits, jnp.float32)

    m = m_ref[...]
    inv_s = 1.0 / s_ref[...]
    zeros = jnp.zeros((e, 12), jnp.float32)
    params_ref[...] = jnp.concatenate([zeros[:, :1], thresh, m, inv_s, zeros],
                                      axis=1)


def _bisect(logits_t, m, s, cap):
    e, n = logits_t.shape
    return pl.pallas_call(
        functools.partial(_bisect_body, cap),
        grid=(1,),
        in_specs=[
            pl.BlockSpec((e, n), lambda i: (0, 0)),
            pl.BlockSpec((e, 1), lambda i: (0, 0)),
            pl.BlockSpec((e, 1), lambda i: (0, 0)),
        ],
        out_specs=pl.BlockSpec((e, 16), lambda i: (0, 0)),
        out_shape=jax.ShapeDtypeStruct((e, 16), jnp.float32),
    )(logits_t, m, s)


def _splat_lane(ref, lane):
    return plsc.load_gather(ref, [jnp.full((16,), lane, jnp.int32)])


def _sc_compact_body(logits_hbm, params_hbm, x_hbm,
                     topidx_hbm, topprob_hbm, expertin_hbm,
                     chunk_v, idx_v, prob_v, tieidx_v, tieprob_v,
                     param_v, rows_v, sem, dsem):
    wid = lax.axis_index("s") * SC_CORES + lax.axis_index("c")
    lanes = jax.lax.iota(jnp.int32, 16)
    n_iters = LOGIT_CHUNK // 16

    for t in range(EXP_PER_WORKER):
        e = wid * EXP_PER_WORKER + t
        pltpu.sync_copy(params_hbm.at[e], param_v)
        thresh = _splat_lane(param_v, 1)
        m_e = _splat_lane(param_v, 2)
        inv_s = _splat_lane(param_v, 3)

        def chunk_loop(ci, carry):
            cnt, tie = carry

            def vec_loop(j, carry2):
                cnt2, tie2 = carry2
                v = chunk_v[pl.ds(j * 16, 16)]
                base = ci * LOGIT_CHUNK + j * 16
                idxv = lanes + base
                gt = v > thresh
                eq = v == thresh
                prob = jnp.exp(v - m_e) * inv_s
                gcs = plsc.cumsum(jnp.where(gt, 1, 0))
                pos = cnt2 + gcs - 1
                plsc.store_scatter(idx_v, [pos], idxv, mask=gt)
                plsc.store_scatter(prob_v, [pos], prob, mask=gt)
                ecs = plsc.cumsum(jnp.where(eq, 1, 0))
                tpos = tie2 + ecs - 1
                teq = eq & (tpos < CAP)
                plsc.store_scatter(tieidx_v, [tpos], idxv, mask=teq)
                plsc.store_scatter(tieprob_v, [tpos], prob, mask=teq)
                cnt2 = cnt2 + plsc.all_reduce_population_count(gt)
                tie2 = tie2 + plsc.all_reduce_population_count(eq)
                return cnt2, tie2

            pltpu.sync_copy(
                logits_hbm.at[pl.ds(e * N_TOKENS + ci * LOGIT_CHUNK,
                                    LOGIT_CHUNK)], chunk_v)
            return jax.lax.fori_loop(0, n_iters, vec_loop, (cnt, tie))

        zero16 = jnp.zeros((16,), jnp.int32)
        cnt, _tie = jax.lax.fori_loop(0, N_TOKENS // LOGIT_CHUNK, chunk_loop,
                                      (zero16, zero16))

        take = CAP - cnt

        def tie_loop(k, _):
            gl = k * 16 + lanes
            msk = gl < take
            tv = tieidx_v[pl.ds(k * 16, 16)]
            tp = tieprob_v[pl.ds(k * 16, 16)]
            pos = cnt + gl
            plsc.store_scatter(idx_v, [pos], tv, mask=msk)
            plsc.store_scatter(prob_v, [pos], tp, mask=msk)
            return 0

        jax.lax.fori_loop(0, CAP // 16, tie_loop, 0)

        pltpu.sync_copy(idx_v, topidx_hbm.at[pl.ds(e * CAP, CAP)])
        pltpu.sync_copy(prob_v, topprob_hbm.at[pl.ds(e * CAP, CAP)])

        def gather_loop(g, _):
            idx_ref = idx_v.at[pl.ds(g * GATHER_CHUNK, GATHER_CHUNK)]
            pltpu.async_copy(x_hbm.at[idx_ref], rows_v, sem).wait()
            pltpu.sync_copy(
                rows_v,
                expertin_hbm.at[pl.ds(e * CAP + g * GATHER_CHUNK,
                                      GATHER_CHUNK)])
            return 0

        jax.lax.fori_loop(0, CAP // GATHER_CHUNK, gather_loop, 0)


def _sc_compact(logits_t, params, x):
    n, h = x.shape
    mesh = plsc.VectorSubcoreMesh(core_axis_name="c", subcore_axis_name="s",
                                  num_cores=SC_CORES,
                                  num_subcores=SC_SUBCORES)
    f = pl.kernel(
        _sc_compact_body,
        out_type=[
            jax.ShapeDtypeStruct((NUM_EXPERTS * CAP,), jnp.int32),
            jax.ShapeDtypeStruct((NUM_EXPERTS * CAP,), jnp.float32),
            jax.ShapeDtypeStruct((NUM_EXPERTS * CAP, h), jnp.float32),
        ],
        mesh=mesh,
        compiler_params=pltpu.CompilerParams(needs_layout_passes=False),
        scratch_types=[
            pltpu.VMEM((LOGIT_CHUNK,), jnp.float32),
            pltpu.VMEM((CAP,), jnp.int32),
            pltpu.VMEM((CAP,), jnp.float32),
            pltpu.VMEM((CAP,), jnp.int32),
            pltpu.VMEM((CAP,), jnp.float32),
            pltpu.VMEM((16,), jnp.float32),
            pltpu.VMEM((GATHER_CHUNK, h), jnp.float32),
            pltpu.SemaphoreType.DMA,
            pltpu.SemaphoreType.DMA,
        ],
    )
    return f(logits_t.reshape(-1), params, x)


def _ffn_body(nk, xe_ref, wg_ref, wu_ref, wd_ref, p_ref, out_ref):
    k = pl.program_id(1)
    xe = xe_ref[0].astype(jnp.bfloat16)
    wg = wg_ref[0].astype(jnp.bfloat16)
    wu = wu_ref[0].astype(jnp.bfloat16)
    g = jax.lax.dot_general(xe, wg, (((1,), (1,)), ((), ())),
                            preferred_element_type=jnp.float32)
    u = jax.lax.dot_general(xe, wu, (((1,), (1,)), ((), ())),
                            preferred_element_type=jnp.float32)
    hact = ((g * jax.nn.sigmoid(g)) * u).astype(jnp.bfloat16)
    wd = wd_ref[0].astype(jnp.bfloat16)
    o = jax.lax.dot_general(hact, wd, (((1,), (1,)), ((), ())),
                            preferred_element_type=jnp.float32)

    @pl.when(k == 0)
    def _():
        out_ref[0] = o

    @pl.when(k > 0)
    def _():
        out_ref[0] += o

    @pl.when(k == nk - 1)
    def _():
        out_ref[0] *= p_ref[0]


def _ffn(expert_in, gate_proj_w, up_proj_w, down_proj_w, top_probs):
    e, cap, h = expert_in.shape
    inter = gate_proj_w.shape[1]
    nk = inter // FFN_IC
    return pl.pallas_call(
        functools.partial(_ffn_body, nk),
        grid=(e, nk),
        in_specs=[
            pl.BlockSpec((1, cap, h), lambda ei, k: (ei, 0, 0)),
            pl.BlockSpec((1, FFN_IC, h), lambda ei, k: (ei, k, 0)),
            pl.BlockSpec((1, FFN_IC, h), lambda ei, k: (ei, k, 0)),
            pl.BlockSpec((1, h, FFN_IC), lambda ei, k: (ei, 0, k)),
            pl.BlockSpec((1, cap, 1), lambda ei, k: (ei, 0, 0)),
        ],
        out_specs=pl.BlockSpec((1, cap, h), lambda ei, k: (ei, 0, 0)),
        out_shape=jax.ShapeDtypeStruct((e, cap, h), jnp.float32),
    )(expert_in, gate_proj_w, up_proj_w, down_proj_w, top_probs[..., None])


def kernel(hidden_states, gate_w, gate_proj_w, up_proj_w, down_proj_w):
    b, seq, h = hidden_states.shape
    x = hidden_states.reshape(-1, h)
    n = x.shape[0]
    e = gate_w.shape[0]
    cap = max(n // e, 1)
    cap = min(cap, n)

    logits_t, m, s, aux_sum = _router(x, gate_w)
    params = _bisect(logits_t, m, s, cap)
    flat_idx, flat_probs, expert_in_flat = _sc_compact(logits_t, params, x)

    top_idx = flat_idx.reshape(e, cap)
    top_probs = flat_probs.reshape(e, cap)
    expert_in = expert_in_flat.reshape(e, cap, h)

    weighted = _ffn(expert_in, gate_proj_w, up_proj_w, down_proj_w, top_probs)

    final = jnp.zeros_like(x).at[flat_idx].add(weighted.reshape(-1, h))
    token_counts = jnp.zeros((n,), x.dtype).at[flat_idx].add(flat_probs)
    token_counts = jnp.clip(token_counts, 1e-9, None)
    final = (final / token_counts[:, None]).reshape(b, seq, h)

    aux_loss = (aux_sum.reshape(()) / n) * 0.001
    return final, aux_loss
